# baseline (speedup 1.0000x reference)
import jax
import jax.numpy as jnp
from jax.experimental import pallas as pl
from jax.experimental.pallas import tpu as pltpu

_ROWS = 128
_COLS = 32768
_TOTAL = _ROWS * _COLS
_KK = 64 * _ROWS
_CH = 8
_NCH = _ROWS // _CH
_SUB = 4096
_NSUB = _COLS // _SUB

_i32 = jnp.int32
_f32 = jnp.float32


def _lt(k, t):
    return jax.lax.shift_right_logical(k - t, 31)


def _body(x_ref, o_ref):
    for c in range(_NCH):
        xb = x_ref[c * _CH:(c + 1) * _CH, :]
        keys = jnp.maximum(jax.lax.bitcast_convert_type(xb, _i32), 0)
        o_ref[c * _CH:(c + 1) * _CH, :] = jax.lax.bitcast_convert_type(
            keys, _f32
        )

    def kvreg(c, s):
        return jax.lax.bitcast_convert_type(
            o_ref[c * _CH:(c + 1) * _CH, s * 128:(s + 1) * 128], _i32
        )

    def kchunk(c):
        return jax.lax.bitcast_convert_type(
            o_ref[c * _CH:(c + 1) * _CH, :], _i32
        )

    _NV = _COLS // 128

    def count_lt(t):
        accs = [jnp.zeros((_CH, 128), _i32) for _ in range(8)]
        i = 0
        for c in range(_NCH):
            for s in range(_NV):
                accs[i & 7] = accs[i & 7] + _lt(kvreg(c, s), t)
                i += 1
        a = accs
        while len(a) > 1:
            a = [a[j] + a[j + 1] for j in range(0, len(a), 2)]
        return jnp.sum(a[0])

    kk = jnp.int32(_KK)
    ge_kk = jnp.int32(_TOTAL - _KK)

    def key_round(i, cur):
        cand = cur + (jnp.int32(1) << (jnp.int32(30) - i))
        return jnp.where(count_lt(cand) <= ge_kk, cand, cur)

    kstar = jax.lax.fori_loop(0, 31, key_round, jnp.int32(0))

    gaccs = [jnp.zeros((_CH, 128), _i32) for _ in range(8)]
    rows = []
    for c in range(_NCH):
        raccs = [jnp.zeros((_CH, 128), _i32) for _ in range(4)]
        for s in range(_NV):
            k = kvreg(c, s)
            le = _lt(k, kstar + 1)
            gaccs[s & 7] = gaccs[s & 7] + le
            raccs[s & 3] = raccs[s & 3] + (le - _lt(k, kstar))
        racc = (raccs[0] + raccs[1]) + (raccs[2] + raccs[3])
        rows.append(jnp.sum(racc, axis=1, keepdims=True))
    ga = gaccs
    while len(ga) > 1:
        ga = [ga[j] + ga[j + 1] for j in range(0, len(ga), 2)]
    count_gt = jnp.int32(_TOTAL) - jnp.sum(ga[0])
    rc = jnp.concatenate(rows, axis=0)
    r = kk - count_gt

    row_iota = jax.lax.broadcasted_iota(_i32, (_ROWS, 1), 0)

    def row_prefix(a):
        return jnp.sum(jnp.where(row_iota < a, rc, 0))

    def row_round(i, lo):
        cand = lo + (jnp.int32(64) >> i)
        return jnp.where(row_prefix(cand) < r, cand, lo)

    brow = jax.lax.fori_loop(0, 7, row_round, jnp.int32(0))
    rem = r - row_prefix(brow)

    eq_row = (
        jax.lax.bitcast_convert_type(o_ref[pl.ds(brow, 1), :], _i32) == kstar
    ).astype(_i32)
    col_iota = jax.lax.broadcasted_iota(_i32, (1, _COLS), 1)

    def col_prefix(c):
        return jnp.sum(jnp.where(col_iota < c, eq_row, 0))

    def col_round(i, lo):
        cand = lo + (jnp.int32(16384) >> i)
        return jnp.where(col_prefix(cand) < rem, cand, lo)

    locol = jax.lax.fori_loop(0, 15, col_round, jnp.int32(0))

    cut = jnp.where(
        row_iota < brow,
        jnp.int32(_COLS),
        jnp.where(row_iota == brow, locol + 1, jnp.int32(0)),
    )

    for c in range(_NCH):
        k = kchunk(c)
        cid = jax.lax.broadcasted_iota(_i32, (_CH, _COLS), 1)
        cutc = cut[c * _CH:(c + 1) * _CH, :]
        keep = (k > kstar) | ((k == kstar) & (cid < cutc))
        o_ref[c * _CH:(c + 1) * _CH, :] = jnp.where(
            keep, jax.lax.bitcast_convert_type(k, _f32), 0.0
        )


def kernel(x):
    return pl.pallas_call(
        _body,
        out_shape=jax.ShapeDtypeStruct((_ROWS, _COLS), jnp.float32),
        in_specs=[pl.BlockSpec((_ROWS, _COLS), lambda: (0, 0))],
        out_specs=pl.BlockSpec((_ROWS, _COLS), lambda: (0, 0)),
    )(x)
